# baseline (speedup 1.0000x reference)
import functools

import jax
import jax.numpy as jnp
from jax import lax
from jax.experimental import pallas as pl
from jax.experimental.pallas import tpu as pltpu
from jax.experimental.pallas import tpu_sc as plsc

NUM_NODES = 1000000
NODE_DIM = 64
HIDDEN = 128
B = 16384

NC = 2
NS = 16
NW = NC * NS
B_PER_W = B // NW
CHUNK = 128
NCHUNK = B_PER_W // CHUNK
NBUF = 4


def _gather_body(ws_hbm, d_hbm, src_hbm, dst_hbm, out_hbm,
                 src_i, dst_i, bufs, sems):
    wid = lax.axis_index("s") * NC + lax.axis_index("c")
    base = wid * B_PER_W
    pltpu.sync_copy(src_hbm.at[wid], src_i)
    pltpu.sync_copy(dst_hbm.at[wid], dst_i)

    tasks = []
    for j in range(NCHUNK):
        tasks.append((ws_hbm, src_i, j, 0))
        tasks.append((ws_hbm, dst_i, j, 1))
        tasks.append((d_hbm, src_i, j, 2))
        tasks.append((d_hbm, dst_i, j, 3))

    copies = [None] * NBUF
    pending = [None] * NBUF
    for t, (table, idxr, j, g) in enumerate(tasks):
        b = t % NBUF
        if copies[b] is not None:
            copies[b].wait()
            pg, pj = pending[b]
            pltpu.sync_copy(bufs[b], out_hbm.at[pg, pl.ds(base + pj * CHUNK, CHUNK)])
        copies[b] = pltpu.async_copy(table.at[idxr.at[j]], bufs[b], sems[b])
        pending[b] = (g, j)
    for b in range(NBUF):
        copies[b].wait()
        pg, pj = pending[b]
        pltpu.sync_copy(bufs[b], out_hbm.at[pg, pl.ds(base + pj * CHUNK, CHUNK)])


def _sc_gather(W_static, D, src_g, dst_g):
    mesh = plsc.VectorSubcoreMesh(core_axis_name="c", subcore_axis_name="s",
                                  num_cores=NC, num_subcores=NS)
    scratch = (
        [pltpu.VMEM((NCHUNK, CHUNK), jnp.int32)] * 2
        + [pltpu.VMEM((CHUNK, NODE_DIM), jnp.float32)] * NBUF
        + [pltpu.SemaphoreType.DMA] * NBUF
    )

    def body(ws_hbm, d_hbm, src_hbm, dst_hbm, out_hbm, *rest):
        src_i, dst_i = rest[0], rest[1]
        bufs = list(rest[2:2 + NBUF])
        sems = list(rest[2 + NBUF:])
        _gather_body(ws_hbm, d_hbm, src_hbm, dst_hbm, out_hbm,
                     src_i, dst_i, bufs, sems)

    k = pl.kernel(
        body,
        out_type=jax.ShapeDtypeStruct((4, B, NODE_DIM), jnp.float32),
        mesh=mesh,
        scratch_types=scratch,
        compiler_params=pltpu.CompilerParams(use_tc_tiling_on_sc=False),
    )
    return k(W_static, D, src_g, dst_g)


def _mlp_body(hp_ref, w1t_ref, b1_ref, w2t_ref, b2_ref, out_ref):
    acc = jnp.dot(hp_ref[0], w1t_ref[0:64, :], preferred_element_type=jnp.float32)
    acc += jnp.dot(hp_ref[1], w1t_ref[64:128, :], preferred_element_type=jnp.float32)
    acc += jnp.dot(hp_ref[2], w1t_ref[128:192, :], preferred_element_type=jnp.float32)
    acc += jnp.dot(hp_ref[3], w1t_ref[192:256, :], preferred_element_type=jnp.float32)
    h1 = jnp.maximum(acc + b1_ref[...], 0.0)
    out_ref[...] = jnp.dot(h1, w2t_ref[...], preferred_element_type=jnp.float32) + b2_ref[...]


def _tc_mlp(hparts, W1, b1, W2, b2):
    blk = 2048
    grid = (B // blk,)
    w1t = W1.T
    w2t = W2.T
    out = pl.pallas_call(
        _mlp_body,
        grid=grid,
        in_specs=[
            pl.BlockSpec((4, blk, NODE_DIM), lambda i: (0, i, 0)),
            pl.BlockSpec((256, HIDDEN), lambda i: (0, 0)),
            pl.BlockSpec((1, HIDDEN), lambda i: (0, 0)),
            pl.BlockSpec((HIDDEN, 1), lambda i: (0, 0)),
            pl.BlockSpec((1, 1), lambda i: (0, 0)),
        ],
        out_specs=pl.BlockSpec((blk, 1), lambda i: (i, 0)),
        out_shape=jax.ShapeDtypeStruct((B, 1), jnp.float32),
    )(hparts, w1t, b1.reshape(1, HIDDEN), w2t, b2.reshape(1, 1))
    return out.reshape(B)


def kernel(src, dst, ts, W_static, D, W1, b1, W2, b2):
    src_g = src.astype(jnp.int32).reshape(NW, NCHUNK, CHUNK)
    dst_g = dst.astype(jnp.int32).reshape(NW, NCHUNK, CHUNK)
    hparts = _sc_gather(W_static, D, src_g, dst_g)
    return _tc_mlp(hparts, W1, b1, W2, b2)
